# baseline (speedup 1.0000x reference)
import numpy as np
import jax
import jax.numpy as jnp
from jax import lax
from jax.experimental import pallas as pl
from jax.experimental.pallas import tpu as pltpu
from jax.experimental.pallas import tpu_sc as plsc

_VOCAB = 100000
_D = 1024
_B = 4
_S = 8192
_N = _B * _S
_NC, _NS = 2, 16
_NW = _NC * _NS
_PPW = _S // _NW
_C = 8
_CHUNKS = _PPW // _C
_NBUF = 3
_LANES = 16


def _make_pe():
    pos = np.arange(_S, dtype=np.float32)[:, None]
    i = np.arange(0, _D, 2, dtype=np.float32)
    div = np.exp(-(np.log(10000.0)) * i / np.float32(_D)).astype(np.float32)
    ang = pos * div[None, :]
    pe = np.zeros((_S, _D), dtype=np.float32)
    pe[:, 0::2] = np.sin(ang)
    pe[:, 1::2] = np.cos(ang)
    return pe


_PE = _make_pe()


def _body(x_hbm, pe_hbm, tab_hbm, out_hbm, idx_all, pe_v, rows,
          gsem0, gsem1, gsem2, ssem0, ssem1, ssem2, psem0, psem1, psem2):
    c = lax.axis_index("c")
    s = lax.axis_index("s")
    wid = s * _NC + c
    p0 = wid * _PPW
    gsem = (gsem0, gsem1, gsem2)
    ssem = (ssem0, ssem1, ssem2)
    psem = (psem0, psem1, psem2)

    def launch_chunk(k, p):
        pltpu.async_copy(pe_hbm.at[pl.ds(p0 + k * _C, _C)], pe_v.at[p],
                         psem[p])
        for b in range(_B):
            pltpu.async_copy(
                tab_hbm.at[idx_all.at[b, pl.ds(k * _C, _C)]],
                rows.at[p, b], gsem[p])

    def drain_stores(p):
        for _ in range(_B):
            pltpu.make_async_copy(rows.at[p, 0], out_hbm.at[pl.ds(0, _C)],
                                  ssem[p]).wait()

    def chunk_step(i, k, p, first, last):
        s0 = p0 + k * _C
        pn = (p + 2) % _NBUF
        pltpu.make_async_copy(pe_hbm.at[pl.ds(0, _C)], pe_v.at[p],
                              psem[p]).wait()
        for _ in range(_B):
            pltpu.make_async_copy(
                tab_hbm.at[idx_all.at[0, pl.ds(0, _C)]], rows.at[p, 0],
                gsem[p]).wait()
        if first:
            if i is not None:
                @pl.when(i > 0)
                def _():
                    drain_stores(pn)
        else:
            drain_stores(pn)
        if not last:
            launch_chunk(k + 2, pn)

        def add_row(r, carry2):
            for v in range(_D // _LANES):
                sl = pl.ds(v * _LANES, _LANES)
                pe = pe_v[p, r, sl]
                for b in range(_B):
                    rows[p, b, r, sl] = rows[p, b, r, sl] + pe
            return carry2
        lax.fori_loop(0, _C, add_row, 0)

        for b in range(_B):
            pltpu.async_copy(rows.at[p, b],
                             out_hbm.at[pl.ds(b * _S + s0, _C)], ssem[p])

    for b in range(_B):
        pltpu.sync_copy(x_hbm.at[pl.ds(b * _S + p0, _PPW)], idx_all.at[b])

    launch_chunk(0, 0)
    launch_chunk(1, 1)

    def tri_body(i, carry):
        for j in range(_NBUF):
            chunk_step(i, _NBUF * i + j, j, first=(j == 0), last=False)
        return carry

    lax.fori_loop(0, (_CHUNKS - 2) // _NBUF, tri_body, 0)

    chunk_step(None, _CHUNKS - 2, 0, first=False, last=True)
    chunk_step(None, _CHUNKS - 1, 1, first=False, last=True)
    drain_stores(1)


def kernel(x, tok_table):
    x_flat = x.reshape(_N)
    mesh = plsc.VectorSubcoreMesh(
        core_axis_name="c", subcore_axis_name="s",
        num_cores=_NC, num_subcores=_NS)
    f = pl.kernel(
        _body,
        out_type=jax.ShapeDtypeStruct((_N, _D), jnp.float32),
        mesh=mesh,
        scratch_types=[
            pltpu.VMEM((_B, _PPW), jnp.int32),
            pltpu.VMEM((_NBUF, _C, _D), jnp.float32),
            pltpu.VMEM((_NBUF, _B, _C, _D), jnp.float32),
        ] + [pltpu.SemaphoreType.DMA] * 9,
    )
    out = f(x_flat, _PE, tok_table)
    return out.reshape(_B, _S, _D)


# speedup vs baseline: 1.1258x; 1.1258x over previous
import numpy as np
import jax
import jax.numpy as jnp
from jax import lax
from jax.experimental import pallas as pl
from jax.experimental.pallas import tpu as pltpu
from jax.experimental.pallas import tpu_sc as plsc

_VOCAB = 100000
_D = 1024
_B = 4
_S = 8192
_N = _B * _S
_NC, _NS = 2, 16
_NW = _NC * _NS
_PPW = _S // _NW
_C = 8
_BC = _B * _C
_CHUNKS = _PPW // _C
_NPAIR = _CHUNKS // 2
_LANES = 16
_DW = _D // 2


def _bf16_bits(a):
    u = a.astype(np.float32).view(np.uint32).astype(np.uint64)
    return ((u + 0x7FFF + ((u >> 16) & 1)) >> 16).astype(np.uint32)


def _make_pe_packed():
    pos = np.arange(_S, dtype=np.float32)[:, None]
    i = np.arange(0, _D, 2, dtype=np.float32)
    div = np.exp(-(np.log(10000.0)) * i / np.float32(_D)).astype(np.float32)
    ang = pos * div[None, :]
    pe = np.zeros((_S, _D), dtype=np.float32)
    pe[:, 0::2] = np.sin(ang)
    pe[:, 1::2] = np.cos(ang)
    pe3 = pe.reshape(_S, _D // 32, 2, 16)
    words = _bf16_bits(pe3[:, :, 0, :]) | (_bf16_bits(pe3[:, :, 1, :]) << 16)
    return words.reshape(_S, _DW).view(np.int32)


_PE = _make_pe_packed()


def _body(x2_hbm, pe_hbm, tab_hbm, out_hbm, idx_all, pe_v, rows,
          gsem0, gsem1, ssem0, ssem1, pesem):
    c = lax.axis_index("c")
    s = lax.axis_index("s")
    wid = s * _NC + c
    p0 = wid * _PPW
    gsem = (gsem0, gsem1)
    ssem = (ssem0, ssem1)

    def drain(n, src, dst, sem):
        for _ in range(n):
            pltpu.make_async_copy(src, dst, sem).wait()

    pltpu.sync_copy(x2_hbm.at[wid], idx_all)

    pltpu.async_copy(pe_hbm.at[pl.ds(p0, _C)], pe_v.at[0], pesem)
    pltpu.async_copy(tab_hbm.at[idx_all.at[0]], rows.at[0], gsem0)

    def pair_body(i, carry):
        for half in (0, 1):
            k = 2 * i + half
            s0 = p0 + k * _C
            pltpu.make_async_copy(
                pe_hbm.at[pl.ds(0, _C)], pe_v.at[half], pesem).wait()
            if half == 0:
                pltpu.async_copy(pe_hbm.at[pl.ds(s0 + _C, _C)],
                                 pe_v.at[1], pesem)
            else:
                @pl.when(i < _NPAIR - 1)
                def _():
                    pltpu.async_copy(pe_hbm.at[pl.ds(s0 + _C, _C)],
                                     pe_v.at[0], pesem)
            drain(1, tab_hbm.at[idx_all.at[0]], rows.at[half], gsem[half])
            if half == 0:
                @pl.when(i > 0)
                def _():
                    drain(_B, rows.at[1, pl.ds(0, _C)],
                          out_hbm.at[pl.ds(0, _C)], ssem1)
            else:
                drain(_B, rows.at[0, pl.ds(0, _C)],
                      out_hbm.at[pl.ds(0, _C)], ssem0)
            if half == 0:
                pltpu.async_copy(tab_hbm.at[idx_all.at[k + 1]],
                                 rows.at[1], gsem1)
            else:
                @pl.when(i < _NPAIR - 1)
                def _():
                    pltpu.async_copy(tab_hbm.at[idx_all.at[k + 1]],
                                     rows.at[0], gsem0)

            def add_row(r, carry2):
                for g in range(_D // 32):
                    w = pe_v[half, r, pl.ds(g * _LANES, _LANES)]
                    sh = jnp.full((_LANES,), 16, dtype=jnp.int32)
                    msk = jnp.full((_LANES,), -65536, dtype=jnp.int32)
                    fa = lax.bitcast_convert_type(
                        lax.shift_left(w, sh), jnp.float32)
                    fb = lax.bitcast_convert_type(
                        lax.bitwise_and(w, msk), jnp.float32)
                    sla = pl.ds(g * 32, _LANES)
                    slb = pl.ds(g * 32 + _LANES, _LANES)
                    for b in range(_B):
                        j = b * _C
                        rows[half, j + r, sla] = rows[half, j + r, sla] + fa
                        rows[half, j + r, slb] = rows[half, j + r, slb] + fb
                return carry2
            lax.fori_loop(0, _C, add_row, 0)

            for b in range(_B):
                pltpu.async_copy(rows.at[half, pl.ds(b * _C, _C)],
                                 out_hbm.at[pl.ds(b * _S + s0, _C)],
                                 ssem[half])
        return carry

    lax.fori_loop(0, _NPAIR, pair_body, 0)

    drain(_B, rows.at[1, pl.ds(0, _C)], out_hbm.at[pl.ds(0, _C)], ssem1)


def kernel(x, tok_table):
    x2 = (x.reshape(_B, _NW, _CHUNKS, _C)
           .transpose(1, 2, 0, 3)
           .reshape(_NW, _CHUNKS, _BC))
    mesh = plsc.VectorSubcoreMesh(
        core_axis_name="c", subcore_axis_name="s",
        num_cores=_NC, num_subcores=_NS)
    f = pl.kernel(
        _body,
        out_type=jax.ShapeDtypeStruct((_N, _D), jnp.float32),
        mesh=mesh,
        scratch_types=[
            pltpu.VMEM((_CHUNKS, _BC), jnp.int32),
            pltpu.VMEM((2, _C, _DW), jnp.int32),
            pltpu.VMEM((2, _BC, _D), jnp.float32),
            pltpu.SemaphoreType.DMA,
            pltpu.SemaphoreType.DMA,
            pltpu.SemaphoreType.DMA,
            pltpu.SemaphoreType.DMA,
            pltpu.SemaphoreType.DMA,
        ],
    )
    out = f(x2, _PE, tok_table)
    return out.reshape(_B, _S, _D)
